# baseline (speedup 1.0000x reference)
import jax
import jax.numpy as jnp
from jax import lax
from jax.experimental import pallas as pl
from jax.experimental.pallas import tpu as pltpu

_NUM_SUBJECTS = 100000
_EMBED_DIM = 64
_FEATURE_DIM = 128
_BATCH = 4096
_SEQ_LEN = 50

_CB = 128
_K = 4
_NC = _BATCH // _CB
_GU = 8


def _tc_body(ids_ref, par_ref, w_ref, b_ref, table_hbm, f_hbm, o_hbm,
             erows, ibufs, obufs, gsems, in_sems, out_sems):
    n_groups = _NC // _K

    def in_copy(c, k):
        return pltpu.make_async_copy(
            f_hbm.at[:, pl.ds(c * _CB, _CB), :], ibufs[k], in_sems[k])

    def out_copy(c, k):
        return pltpu.make_async_copy(
            obufs[k], o_hbm.at[:, pl.ds(c * _CB, _CB), :], out_sems[k])

    def gather_issue(c, k):
        def gissue(i, _):
            for j in range(_GU):
                r = c * _CB + i * _GU + j
                rid = ids_ref[r] // 2
                pltpu.make_async_copy(
                    table_hbm.at[pl.ds(rid, 1)], erows.at[pl.ds(r, 1)],
                    gsems[k],
                ).start()
            return 0
        lax.fori_loop(0, _CB // _GU, gissue, 0, unroll=False)

    def gather_wait(c, k):
        pltpu.make_async_copy(
            table_hbm.at[pl.ds(0, _CB)], erows.at[pl.ds(c * _CB, _CB)],
            gsems[k],
        ).wait()

    for k in range(_K):
        in_copy(k, k).start()
        gather_issue(k, k)

    def group(g, _):
        for k in range(_K):
            c = g * _K + k

            @pl.when(g > 0)
            def _():
                out_copy(c - _K, k).wait()

            gather_wait(c, k)

            @pl.when(g + 1 < n_groups)
            def _():
                gather_issue(c + _K, k)

            packed = erows[pl.ds(c * _CB, _CB), :]
            par = par_ref[pl.ds(c * _CB, _CB), :] != 0
            sel = jnp.where(par, packed[:, _EMBED_DIM:], packed[:, :_EMBED_DIM])
            proj = lax.dot_general(
                sel, w_ref[...],
                (((1,), (0,)), ((), ())),
                preferred_element_type=jnp.float32,
            ) + b_ref[...]
            in_copy(c, k).wait()
            obufs[k][...] = ibufs[k][...] + proj[None, :, :]
            out_copy(c, k).start()

            @pl.when(g + 1 < n_groups)
            def _():
                in_copy(c + _K, k).start()
        return 0

    lax.fori_loop(0, n_groups, group, 0, unroll=False)
    for k in range(_K):
        out_copy(_NC - _K + k, k).wait()


def kernel(features, subject_ids, emb_table, W, b):
    ids = subject_ids.astype(jnp.int32)
    ft = jnp.transpose(features, (1, 0, 2))
    wt = jnp.transpose(W, (1, 0))
    scratch = [
        pltpu.VMEM((_BATCH, 2 * _EMBED_DIM), jnp.float32),
        [pltpu.VMEM((_SEQ_LEN, _CB, _FEATURE_DIM), jnp.float32)
         for _ in range(_K)],
        [pltpu.VMEM((_SEQ_LEN, _CB, _FEATURE_DIM), jnp.float32)
         for _ in range(_K)],
        [pltpu.SemaphoreType.DMA for _ in range(_K)],
        [pltpu.SemaphoreType.DMA for _ in range(_K)],
        [pltpu.SemaphoreType.DMA for _ in range(_K)],
    ]
    packed_table = emb_table.reshape(_NUM_SUBJECTS // 2, 2 * _EMBED_DIM)
    parity = (ids % 2).reshape(_BATCH, 1)
    out_t = pl.pallas_call(
        _tc_body,
        in_specs=[
            pl.BlockSpec(memory_space=pltpu.SMEM),
            pl.BlockSpec(memory_space=pltpu.VMEM),
            pl.BlockSpec(memory_space=pltpu.VMEM),
            pl.BlockSpec(memory_space=pltpu.VMEM),
            pl.BlockSpec(memory_space=pl.ANY),
            pl.BlockSpec(memory_space=pl.ANY),
        ],
        out_specs=pl.BlockSpec(memory_space=pl.ANY),
        out_shape=jax.ShapeDtypeStruct((_SEQ_LEN, _BATCH, _FEATURE_DIM), jnp.float32),
        scratch_shapes=scratch,
    )(ids, parity, wt, b.reshape(1, _FEATURE_DIM), packed_table, ft)
    return jnp.transpose(out_t, (1, 0, 2))


# speedup vs baseline: 1.4359x; 1.4359x over previous
import jax
import jax.numpy as jnp
from jax import lax
from jax.experimental import pallas as pl
from jax.experimental.pallas import tpu as pltpu

_NUM_SUBJECTS = 100000
_EMBED_DIM = 64
_FEATURE_DIM = 128
_BATCH = 4096
_SEQ_LEN = 50

_CB = 128
_K = 4
_NC = _BATCH // _CB
_GU = 8


def _tc_body(ids_ref, w_ref, b_ref, table_hbm, f_hbm, o_hbm,
             erows, ibufs, obufs, gsems, in_sems, out_sems):
    n_groups = _NC // _K

    def in_copy(c, k):
        return pltpu.make_async_copy(
            f_hbm.at[:, pl.ds(c * _CB, _CB), :], ibufs[k], in_sems[k])

    def out_copy(c, k):
        return pltpu.make_async_copy(
            obufs[k], o_hbm.at[:, pl.ds(c * _CB, _CB), :], out_sems[k])

    def gather_issue(c, k):
        def gissue(i, _):
            for j in range(_GU):
                r = c * _CB + i * _GU + j
                rid = ids_ref[r]
                pltpu.make_async_copy(
                    table_hbm.at[pl.ds(rid, 1)], erows.at[pl.ds(r, 1)],
                    gsems[k],
                ).start()
            return 0
        lax.fori_loop(0, _CB // _GU, gissue, 0, unroll=False)

    def gather_wait(c, k):
        pltpu.make_async_copy(
            table_hbm.at[pl.ds(0, _CB)], erows.at[pl.ds(c * _CB, _CB)],
            gsems[k],
        ).wait()

    for k in range(_K):
        in_copy(k, k).start()
        gather_issue(k, k)

    def group(g, _):
        for k in range(_K):
            c = g * _K + k

            @pl.when(g > 0)
            def _():
                out_copy(c - _K, k).wait()

            gather_wait(c, k)

            @pl.when(g + 1 < n_groups)
            def _():
                gather_issue(c + _K, k)

            proj = lax.dot_general(
                erows[pl.ds(c * _CB, _CB), :], w_ref[...],
                (((1,), (0,)), ((), ())),
                preferred_element_type=jnp.float32,
            ) + b_ref[...]
            in_copy(c, k).wait()
            obufs[k][...] = ibufs[k][...] + proj[None, :, :]
            out_copy(c, k).start()

            @pl.when(g + 1 < n_groups)
            def _():
                in_copy(c + _K, k).start()
        return 0

    lax.fori_loop(0, n_groups, group, 0, unroll=False)
    for k in range(_K):
        out_copy(_NC - _K + k, k).wait()


def kernel(features, subject_ids, emb_table, W, b):
    ids = subject_ids.astype(jnp.int32)
    ft = jnp.transpose(features, (1, 0, 2))
    wt = jnp.transpose(W, (1, 0))
    scratch = [
        pltpu.VMEM((_BATCH, _EMBED_DIM), jnp.float32),
        [pltpu.VMEM((_SEQ_LEN, _CB, _FEATURE_DIM), jnp.float32)
         for _ in range(_K)],
        [pltpu.VMEM((_SEQ_LEN, _CB, _FEATURE_DIM), jnp.float32)
         for _ in range(_K)],
        [pltpu.SemaphoreType.DMA for _ in range(_K)],
        [pltpu.SemaphoreType.DMA for _ in range(_K)],
        [pltpu.SemaphoreType.DMA for _ in range(_K)],
    ]
    out_t = pl.pallas_call(
        _tc_body,
        in_specs=[
            pl.BlockSpec(memory_space=pltpu.SMEM),
            pl.BlockSpec(memory_space=pltpu.VMEM),
            pl.BlockSpec(memory_space=pltpu.VMEM),
            pl.BlockSpec(memory_space=pl.ANY),
            pl.BlockSpec(memory_space=pl.ANY),
        ],
        out_specs=pl.BlockSpec(memory_space=pl.ANY),
        out_shape=jax.ShapeDtypeStruct((_SEQ_LEN, _BATCH, _FEATURE_DIM), jnp.float32),
        scratch_shapes=scratch,
    )(ids, wt, b.reshape(1, _FEATURE_DIM), emb_table, ft)
    return jnp.transpose(out_t, (1, 0, 2))
